# baseline (speedup 1.0000x reference)
import jax
import jax.numpy as jnp
from jax.experimental import pallas as pl
from jax.experimental.pallas import tpu as pltpu

_N_ROWS = 65536
_N_COLS = 512
_ONES_ROWS = 512
_BLOCK_ROWS = 8192


def _fill_kernel(o_ref):
    i = pl.program_id(0)
    row = jax.lax.broadcasted_iota(jnp.int32, o_ref.shape, 0) + i * _BLOCK_ROWS
    o_ref[...] = (row < _ONES_ROWS).astype(jnp.float32)


def kernel(x):
    return pl.pallas_call(
        _fill_kernel,
        grid=(_N_ROWS // _BLOCK_ROWS,),
        out_specs=pl.BlockSpec((_BLOCK_ROWS, _N_COLS), lambda i: (i, 0)),
        out_shape=jax.ShapeDtypeStruct((_N_ROWS, _N_COLS), x.dtype),
        compiler_params=pltpu.CompilerParams(
            dimension_semantics=("parallel",),
        ),
    )()


# speedup vs baseline: 1.0824x; 1.0824x over previous
import jax
import jax.numpy as jnp
from jax.experimental import pallas as pl
from jax.experimental.pallas import tpu as pltpu

_N_ROWS = 65536
_N_COLS = 512
_ONES_ROWS = 512
_BLOCK_ROWS = 2048


def _fill_kernel(o_ref):
    i = pl.program_id(0)
    row = jax.lax.broadcasted_iota(jnp.int32, o_ref.shape, 0) + i * _BLOCK_ROWS
    o_ref[...] = (row < _ONES_ROWS).astype(jnp.float32)


def kernel(x):
    return pl.pallas_call(
        _fill_kernel,
        grid=(_N_ROWS // _BLOCK_ROWS,),
        out_specs=pl.BlockSpec((_BLOCK_ROWS, _N_COLS), lambda i: (i, 0)),
        out_shape=jax.ShapeDtypeStruct((_N_ROWS, _N_COLS), x.dtype),
        compiler_params=pltpu.CompilerParams(
            dimension_semantics=("parallel",),
        ),
    )()
